# baseline (speedup 1.0000x reference)
import functools

import jax
import jax.numpy as jnp
from jax import lax
from jax.experimental import pallas as pl
from jax.experimental.pallas import tpu as pltpu
from jax.experimental.pallas import tpu_sc as plsc

N = 10000
IN_CH = 128
OUT_CH = 128

NC = 2
NS = 16
CH = 128
LANES = 16

NP = 2
NQ = NC * NP
Q = 2500
ACC_ROWS = 2560
RPT = ACC_ROWS // NS
DROWS = 32
DRPT = DROWS // NS
APAD = 10016
CAP = 48
CBUF = 176


def _proj_body(x_ref, w_ref, a2_ref, h_ref, a_ref):
    h = jnp.dot(x_ref[...], w_ref[...], preferred_element_type=jnp.float32)
    h_ref[...] = h
    a_ref[...] = jnp.dot(h, a2_ref[...], preferred_element_type=jnp.float32)


def _project(x, W, A2):
    blk = 1000
    grid = N // blk
    return pl.pallas_call(
        _proj_body,
        grid=(grid,),
        in_specs=[
            pl.BlockSpec((blk, IN_CH), lambda i: (i, 0)),
            pl.BlockSpec((IN_CH, OUT_CH), lambda i: (0, 0)),
            pl.BlockSpec((OUT_CH, 128), lambda i: (0, 0)),
        ],
        out_specs=[
            pl.BlockSpec((blk, OUT_CH), lambda i: (i, 0)),
            pl.BlockSpec((blk, 128), lambda i: (i, 0)),
        ],
        out_shape=[
            jax.ShapeDtypeStruct((N, OUT_CH), jnp.float32),
            jax.ShapeDtypeStruct((N, 128), jnp.float32),
        ],
    )(x, W, A2)


def _edge_body(h_hbm, asrc_hbm, adst_hbm, srci_hbm, dsti_hbm, zeros_hbm,
               msg_hbm, den_hbm, asrc_v, adst_v, sidx_v, didx_v, csrc_v,
               cdil_v, ce_v, didxb_v, rows_v, denloc_v, dloc_v, dexp_v,
               idv_v, acc_sh, accd_sh, gsem0):
    c = lax.axis_index("c")
    s = lax.axis_index("s")
    nchunk = srci_hbm.shape[1] - 2
    r0 = s * RPT

    pltpu.sync_copy(asrc_hbm, asrc_v)
    pltpu.sync_copy(adst_hbm, adst_v)
    pltpu.sync_copy(srci_hbm.at[s], sidx_v)
    pltpu.sync_copy(dsti_hbm.at[s], didx_v)
    for t in range(DROWS // 16):
        idv_v[pl.ds(t * 16, 16)] = lax.iota(jnp.int32, 16) + t * 16
    zero16 = jnp.zeros((16,), jnp.float32)

    def pass_body(p, _):
        q = NC * p + c
        base = q * Q

        def zinit_body(z, _):
            pltpu.sync_copy(zeros_hbm, acc_sh.at[pl.ds(r0 + z * 32, 32)])
            return 0

        lax.fori_loop(0, RPT // 32, zinit_body, 0)
        pltpu.sync_copy(zeros_hbm.at[pl.ds(0, DRPT)],
                        accd_sh.at[pl.ds(s * DRPT, DRPT)])

        def zero_body(r, _):
            for t in range(8):
                denloc_v[r, pl.ds(t * 16, 16)] = zero16
            return 0

        lax.fori_loop(0, DROWS, zero_body, 0)
        plsc.subcore_barrier()

        iota16 = lax.iota(jnp.int32, 16)
        zero16i = jnp.zeros((16,), jnp.int32)

        def chunk_body(g, _):
            for t in range(CBUF // 16):
                csrc_v[pl.ds(t * 16, 16)] = zero16i
                cdil_v[pl.ds(t * 16, 16)] = (
                    Q + lax.bitwise_and(iota16 + t * 16, 31))

            def group_body(j, off):
                si = sidx_v[g, pl.ds(j * LANES, LANES)]
                di = didx_v[g, pl.ds(j * LANES, LANES)]
                a = (plsc.load_gather(asrc_v, [si])
                     + plsc.load_gather(adst_v, [di]))
                a = jnp.where(a >= 0.0, a, 0.2 * a)
                e16 = jnp.exp(a)
                dil = di - base
                own = (dil >= 0) & (dil < Q)
                dil_d = jnp.where(own, dil, Q + iota16)
                plsc.addupdate_scatter(
                    denloc_v,
                    [lax.shift_right_logical(dil_d, 7),
                     lax.bitwise_and(dil_d, 127)],
                    e16)
                plsc.store_compressed(csrc_v.at[pl.ds(off, 16)], si, mask=own)
                plsc.store_compressed(cdil_v.at[pl.ds(off, 16)], dil, mask=own)
                plsc.store_compressed(ce_v.at[pl.ds(off, 16)], e16, mask=own)
                return off + plsc.all_reduce_population_count(own)[0]

            n_own = lax.fori_loop(0, CH // LANES, group_body, 0)

            def block(k):
                for t in range(CAP // 16):
                    didxb_v[pl.ds(t * 16, 16)] = cdil_v[
                        pl.ds(k * CAP + t * 16, 16)]
                pltpu.async_copy(
                    h_hbm.at[csrc_v.at[pl.ds(k * CAP, CAP)]], rows_v,
                    gsem0).wait()
                for t in range(CAP // 16):
                    ev = ce_v[pl.ds(k * CAP + t * 16, 16)]
                    for l in range(LANES):
                        je = t * 16 + l
                        eb = jnp.full((16,), ev[l], jnp.float32)
                        for kk in range(OUT_CH // 16):
                            rows_v[je, pl.ds(kk * 16, 16)] = (
                                rows_v[je, pl.ds(kk * 16, 16)] * eb)
                pltpu.sync_copy(rows_v, acc_sh.at[didxb_v], add=True)

            block(0)

            @pl.when(n_own > CAP)
            def _():
                block(1)

            @pl.when(n_own > 2 * CAP)
            def _():
                block(2)

            return 0

        lax.fori_loop(0, nchunk, chunk_body, 0)

        pltpu.sync_copy(denloc_v, accd_sh.at[idv_v], add=True)
        plsc.subcore_barrier()

        pltpu.sync_copy(acc_sh.at[pl.ds(r0, RPT)],
                        msg_hbm.at[q, pl.ds(r0, RPT)])
        pltpu.sync_copy(accd_sh, dloc_v)

        def exp_body(b, _):
            for t in range(2):
                o = r0 + b * 32 + t * 16
                dv = dloc_v[lax.shift_right_logical(o, 7),
                            pl.ds(lax.bitwise_and(o, 127), 16)]
                for l in range(LANES):
                    db = jnp.full((16,), dv[l], jnp.float32)
                    for k in range(OUT_CH // 16):
                        dexp_v[t * 16 + l, pl.ds(k * 16, 16)] = db
            pltpu.sync_copy(dexp_v, den_hbm.at[q, pl.ds(r0 + b * 32, 32)])
            return 0

        lax.fori_loop(0, RPT // 32, exp_body, 0)
        plsc.subcore_barrier()
        return 0

    lax.fori_loop(0, NP, pass_body, 0)


def _edge_pass(h, asrc, adst, srci, dsti):
    zeros = jnp.zeros((32, OUT_CH), jnp.float32)
    mesh = plsc.VectorSubcoreMesh(core_axis_name="c", subcore_axis_name="s")
    nchunk = srci.shape[1]
    kern = functools.partial(
        pl.kernel,
        mesh=mesh,
        compiler_params=pltpu.CompilerParams(needs_layout_passes=False),
        out_type=[
            jax.ShapeDtypeStruct((NQ, ACC_ROWS, OUT_CH), jnp.float32),
            jax.ShapeDtypeStruct((NQ, ACC_ROWS, OUT_CH), jnp.float32),
        ],
        scratch_types=[
            pltpu.VMEM((APAD,), jnp.float32),
            pltpu.VMEM((APAD,), jnp.float32),
            pltpu.VMEM((nchunk, CH), jnp.int32),
            pltpu.VMEM((nchunk, CH), jnp.int32),
            pltpu.VMEM((CBUF,), jnp.int32),
            pltpu.VMEM((CBUF,), jnp.int32),
            pltpu.VMEM((CBUF,), jnp.float32),
            pltpu.VMEM((CAP,), jnp.int32),
            pltpu.VMEM((CAP, OUT_CH), jnp.float32),
            pltpu.VMEM((DROWS, 128), jnp.float32),
            pltpu.VMEM((DROWS, 128), jnp.float32),
            pltpu.VMEM((32, OUT_CH), jnp.float32),
            pltpu.VMEM((DROWS,), jnp.int32),
            pltpu.VMEM_SHARED((ACC_ROWS, OUT_CH), jnp.float32),
            pltpu.VMEM_SHARED((DROWS, 128), jnp.float32),
            pltpu.SemaphoreType.DMA,
        ],
    )(_edge_body)
    return kern(h, asrc, adst, srci, dsti, zeros)


def _fin_body(m_ref, d_ref, b_ref, o_ref):
    o_ref[...] = m_ref[...] / (d_ref[...] + 1e-16) + b_ref[...]


def _finalize(msg, den, bias):
    blk = 1024
    rows = msg.shape[0]
    grid = rows // blk
    return pl.pallas_call(
        _fin_body,
        grid=(grid,),
        in_specs=[
            pl.BlockSpec((blk, OUT_CH), lambda i: (i, 0)),
            pl.BlockSpec((blk, OUT_CH), lambda i: (i, 0)),
            pl.BlockSpec((1, OUT_CH), lambda i: (0, 0)),
        ],
        out_specs=pl.BlockSpec((blk, OUT_CH), lambda i: (i, 0)),
        out_shape=jax.ShapeDtypeStruct((rows, OUT_CH), jnp.float32),
    )(msg, den, bias)


def kernel(x, edge_index, W, att_src, att_dst, bias):
    A2 = jnp.zeros((OUT_CH, 128), jnp.float32)
    A2 = A2.at[:, 0].set(att_src[0].astype(jnp.float32))
    A2 = A2.at[:, 1].set(att_dst[0].astype(jnp.float32))

    h, a = _project(x, W, A2)
    asrc = jnp.pad(a[:, 0], (0, APAD - N))
    adst = jnp.pad(a[:, 1], (0, APAD - N))

    ei = edge_index.astype(jnp.int32)
    loop = jnp.arange(N, dtype=jnp.int32)
    e_tot = ei.shape[1] + N
    per = NS * CH
    nchunk = 2 * (-(-e_tot // (2 * per)))
    e_pad = nchunk * per
    src = jnp.concatenate([ei[0], loop, jnp.zeros((e_pad - e_tot,), jnp.int32)])
    dst = jnp.concatenate([ei[1], loop,
                           jnp.full((e_pad - e_tot,), N, jnp.int32)])
    srci = src.reshape(NS, nchunk, CH)
    dsti = dst.reshape(NS, nchunk, CH)
    srci = jnp.pad(srci, ((0, 0), (0, 2), (0, 0)))
    dsti = jnp.pad(dsti, ((0, 0), (0, 2), (0, 0)), constant_values=N)

    msg, den = _edge_pass(h, asrc, adst, srci, dsti)
    out = _finalize(msg.reshape(NQ * ACC_ROWS, OUT_CH),
                    den.reshape(NQ * ACC_ROWS, OUT_CH),
                    bias.reshape(1, OUT_CH))
    return jnp.concatenate(
        [out[i * ACC_ROWS:i * ACC_ROWS + Q] for i in range(NQ)])[:N]


# speedup vs baseline: 1.0006x; 1.0006x over previous
import functools

import jax
import jax.numpy as jnp
from jax import lax
from jax.experimental import pallas as pl
from jax.experimental.pallas import tpu as pltpu
from jax.experimental.pallas import tpu_sc as plsc

N = 10000
IN_CH = 128
OUT_CH = 128

NC = 2
NS = 16
CH = 128
LANES = 16

NP = 2
NQ = NC * NP
Q = 2500
ACC_ROWS = 2560
RPT = ACC_ROWS // NS
DROWS = 32
DRPT = DROWS // NS
APAD = 10016
CAP = 48
CBUF = 176


def _proj_body(x_ref, w_ref, a2_ref, h_ref, a_ref):
    h = jnp.dot(x_ref[...], w_ref[...], preferred_element_type=jnp.float32)
    h_ref[...] = h
    a_ref[...] = jnp.dot(h, a2_ref[...], preferred_element_type=jnp.float32)


def _project(x, W, A2):
    blk = 1000
    grid = N // blk
    return pl.pallas_call(
        _proj_body,
        grid=(grid,),
        in_specs=[
            pl.BlockSpec((blk, IN_CH), lambda i: (i, 0)),
            pl.BlockSpec((IN_CH, OUT_CH), lambda i: (0, 0)),
            pl.BlockSpec((OUT_CH, 128), lambda i: (0, 0)),
        ],
        out_specs=[
            pl.BlockSpec((blk, OUT_CH), lambda i: (i, 0)),
            pl.BlockSpec((blk, 128), lambda i: (i, 0)),
        ],
        out_shape=[
            jax.ShapeDtypeStruct((N, OUT_CH), jnp.float32),
            jax.ShapeDtypeStruct((N, 128), jnp.float32),
        ],
    )(x, W, A2)


def _edge_body(h_hbm, asrc_hbm, adst_hbm, srci_hbm, dsti_hbm, zeros_hbm,
               msg_hbm, den_hbm, asrc_v, adst_v, sidx_v, didx_v, csrc_v,
               cdil_v, ce_v, didxb_v, sidxb_v, rows_v, denloc_v, dloc_v, dexp_v,
               idv_v, acc_sh, accd_sh, gsem0):
    c = lax.axis_index("c")
    s = lax.axis_index("s")
    nchunk = srci_hbm.shape[1] - 2
    r0 = s * RPT

    pltpu.sync_copy(asrc_hbm, asrc_v)
    pltpu.sync_copy(adst_hbm, adst_v)
    pltpu.sync_copy(srci_hbm.at[s], sidx_v)
    pltpu.sync_copy(dsti_hbm.at[s], didx_v)
    for t in range(DROWS // 16):
        idv_v[pl.ds(t * 16, 16)] = lax.iota(jnp.int32, 16) + t * 16
    zero16 = jnp.zeros((16,), jnp.float32)

    def pass_body(p, _):
        q = NC * p + c
        base = q * Q

        def zinit_body(z, _):
            pltpu.sync_copy(zeros_hbm, acc_sh.at[pl.ds(r0 + z * 32, 32)])
            return 0

        lax.fori_loop(0, RPT // 32, zinit_body, 0)
        pltpu.sync_copy(zeros_hbm.at[pl.ds(0, DRPT)],
                        accd_sh.at[pl.ds(s * DRPT, DRPT)])

        def zero_body(r, _):
            for t in range(8):
                denloc_v[r, pl.ds(t * 16, 16)] = zero16
            return 0

        lax.fori_loop(0, DROWS, zero_body, 0)
        plsc.subcore_barrier()

        iota16 = lax.iota(jnp.int32, 16)
        zero16i = jnp.zeros((16,), jnp.int32)

        def chunk_body(g, _):
            for t in range(CBUF // 16):
                csrc_v[pl.ds(t * 16, 16)] = zero16i
                cdil_v[pl.ds(t * 16, 16)] = (
                    Q + lax.bitwise_and(iota16 + t * 16, 31))

            def group_body(j, off):
                si = sidx_v[g, pl.ds(j * LANES, LANES)]
                di = didx_v[g, pl.ds(j * LANES, LANES)]
                a = (plsc.load_gather(asrc_v, [si])
                     + plsc.load_gather(adst_v, [di]))
                a = jnp.where(a >= 0.0, a, 0.2 * a)
                e16 = jnp.exp(a)
                dil = di - base
                own = (dil >= 0) & (dil < Q)
                dil_d = jnp.where(own, dil, Q + iota16)
                plsc.addupdate_scatter(
                    denloc_v,
                    [lax.shift_right_logical(dil_d, 7),
                     lax.bitwise_and(dil_d, 127)],
                    e16)
                plsc.store_compressed(csrc_v.at[pl.ds(off, 16)], si, mask=own)
                plsc.store_compressed(cdil_v.at[pl.ds(off, 16)], dil, mask=own)
                plsc.store_compressed(ce_v.at[pl.ds(off, 16)], e16, mask=own)
                return off + plsc.all_reduce_population_count(own)[0]

            n_own = lax.fori_loop(0, CH // LANES, group_body, 0)

            def block(k):
                for t in range(CAP // 16):
                    didxb_v[pl.ds(t * 16, 16)] = cdil_v[
                        pl.ds(k * CAP + t * 16, 16)]
                    sidxb_v[pl.ds(t * 16, 16)] = csrc_v[
                        pl.ds(k * CAP + t * 16, 16)]
                pltpu.async_copy(
                    h_hbm.at[sidxb_v], rows_v, gsem0).wait()
                for t in range(CAP // 16):
                    ev = ce_v[pl.ds(k * CAP + t * 16, 16)]
                    for l in range(LANES):
                        je = t * 16 + l
                        eb = jnp.full((16,), ev[l], jnp.float32)
                        for kk in range(OUT_CH // 16):
                            rows_v[je, pl.ds(kk * 16, 16)] = (
                                rows_v[je, pl.ds(kk * 16, 16)] * eb)
                pltpu.sync_copy(rows_v, acc_sh.at[didxb_v], add=True)

            block(0)

            @pl.when(n_own > CAP)
            def _():
                block(1)

            @pl.when(n_own > 2 * CAP)
            def _():
                block(2)

            return 0

        lax.fori_loop(0, nchunk, chunk_body, 0)

        pltpu.sync_copy(denloc_v, accd_sh.at[idv_v], add=True)
        plsc.subcore_barrier()

        pltpu.sync_copy(acc_sh.at[pl.ds(r0, RPT)],
                        msg_hbm.at[q, pl.ds(r0, RPT)])
        pltpu.sync_copy(accd_sh, dloc_v)

        def exp_body(b, _):
            for t in range(2):
                o = r0 + b * 32 + t * 16
                dv = dloc_v[lax.shift_right_logical(o, 7),
                            pl.ds(lax.bitwise_and(o, 127), 16)]
                for l in range(LANES):
                    db = jnp.full((16,), dv[l], jnp.float32)
                    for k in range(OUT_CH // 16):
                        dexp_v[t * 16 + l, pl.ds(k * 16, 16)] = db
            pltpu.sync_copy(dexp_v, den_hbm.at[q, pl.ds(r0 + b * 32, 32)])
            return 0

        lax.fori_loop(0, RPT // 32, exp_body, 0)
        plsc.subcore_barrier()
        return 0

    lax.fori_loop(0, NP, pass_body, 0)


def _edge_pass(h, asrc, adst, srci, dsti):
    zeros = jnp.zeros((32, OUT_CH), jnp.float32)
    mesh = plsc.VectorSubcoreMesh(core_axis_name="c", subcore_axis_name="s")
    nchunk = srci.shape[1]
    kern = functools.partial(
        pl.kernel,
        mesh=mesh,
        compiler_params=pltpu.CompilerParams(needs_layout_passes=False),
        out_type=[
            jax.ShapeDtypeStruct((NQ, ACC_ROWS, OUT_CH), jnp.float32),
            jax.ShapeDtypeStruct((NQ, ACC_ROWS, OUT_CH), jnp.float32),
        ],
        scratch_types=[
            pltpu.VMEM((APAD,), jnp.float32),
            pltpu.VMEM((APAD,), jnp.float32),
            pltpu.VMEM((nchunk, CH), jnp.int32),
            pltpu.VMEM((nchunk, CH), jnp.int32),
            pltpu.VMEM((CBUF,), jnp.int32),
            pltpu.VMEM((CBUF,), jnp.int32),
            pltpu.VMEM((CBUF,), jnp.float32),
            pltpu.VMEM((CAP,), jnp.int32),
            pltpu.VMEM((CAP,), jnp.int32),
            pltpu.VMEM((CAP, OUT_CH), jnp.float32),
            pltpu.VMEM((DROWS, 128), jnp.float32),
            pltpu.VMEM((DROWS, 128), jnp.float32),
            pltpu.VMEM((32, OUT_CH), jnp.float32),
            pltpu.VMEM((DROWS,), jnp.int32),
            pltpu.VMEM_SHARED((ACC_ROWS, OUT_CH), jnp.float32),
            pltpu.VMEM_SHARED((DROWS, 128), jnp.float32),
            pltpu.SemaphoreType.DMA,
        ],
    )(_edge_body)
    return kern(h, asrc, adst, srci, dsti, zeros)


def _fin_body(m_ref, d_ref, b_ref, o_ref):
    o_ref[...] = m_ref[...] / (d_ref[...] + 1e-16) + b_ref[...]


def _finalize(msg, den, bias):
    blk = 1024
    rows = msg.shape[0]
    grid = rows // blk
    return pl.pallas_call(
        _fin_body,
        grid=(grid,),
        in_specs=[
            pl.BlockSpec((blk, OUT_CH), lambda i: (i, 0)),
            pl.BlockSpec((blk, OUT_CH), lambda i: (i, 0)),
            pl.BlockSpec((1, OUT_CH), lambda i: (0, 0)),
        ],
        out_specs=pl.BlockSpec((blk, OUT_CH), lambda i: (i, 0)),
        out_shape=jax.ShapeDtypeStruct((rows, OUT_CH), jnp.float32),
    )(msg, den, bias)


def kernel(x, edge_index, W, att_src, att_dst, bias):
    A2 = jnp.zeros((OUT_CH, 128), jnp.float32)
    A2 = A2.at[:, 0].set(att_src[0].astype(jnp.float32))
    A2 = A2.at[:, 1].set(att_dst[0].astype(jnp.float32))

    h, a = _project(x, W, A2)
    asrc = jnp.pad(a[:, 0], (0, APAD - N))
    adst = jnp.pad(a[:, 1], (0, APAD - N))

    ei = edge_index.astype(jnp.int32)
    loop = jnp.arange(N, dtype=jnp.int32)
    e_tot = ei.shape[1] + N
    per = NS * CH
    nchunk = 2 * (-(-e_tot // (2 * per)))
    e_pad = nchunk * per
    src = jnp.concatenate([ei[0], loop, jnp.zeros((e_pad - e_tot,), jnp.int32)])
    dst = jnp.concatenate([ei[1], loop,
                           jnp.full((e_pad - e_tot,), N, jnp.int32)])
    srci = src.reshape(NS, nchunk, CH)
    dsti = dst.reshape(NS, nchunk, CH)
    srci = jnp.pad(srci, ((0, 0), (0, 2), (0, 0)))
    dsti = jnp.pad(dsti, ((0, 0), (0, 2), (0, 0)), constant_values=N)

    msg, den = _edge_pass(h, asrc, adst, srci, dsti)
    out = _finalize(msg.reshape(NQ * ACC_ROWS, OUT_CH),
                    den.reshape(NQ * ACC_ROWS, OUT_CH),
                    bias.reshape(1, OUT_CH))
    return jnp.concatenate(
        [out[i * ACC_ROWS:i * ACC_ROWS + Q] for i in range(NQ)])[:N]


# speedup vs baseline: 5.4745x; 5.4710x over previous
import functools

import jax
import jax.numpy as jnp
from jax import lax
from jax.experimental import pallas as pl
from jax.experimental.pallas import tpu as pltpu
from jax.experimental.pallas import tpu_sc as plsc

N = 10000
IN_CH = 128
OUT_CH = 128

NC = 2
NS = 16
CH = 128
LANES = 16

NP = 2
NQ = NC * NP
Q = 2500
ACC_ROWS = 2560
RPT = ACC_ROWS // NS
DROWS = 32
DRPT = DROWS // NS
APAD = 10016


def _proj_body(x_ref, w_ref, a2_ref, h_ref, a_ref):
    h = jnp.dot(x_ref[...], w_ref[...], preferred_element_type=jnp.float32)
    h_ref[...] = h
    a_ref[...] = jnp.dot(h, a2_ref[...], preferred_element_type=jnp.float32)


def _project(x, W, A2):
    blk = 1000
    grid = N // blk
    return pl.pallas_call(
        _proj_body,
        grid=(grid,),
        in_specs=[
            pl.BlockSpec((blk, IN_CH), lambda i: (i, 0)),
            pl.BlockSpec((IN_CH, OUT_CH), lambda i: (0, 0)),
            pl.BlockSpec((OUT_CH, 128), lambda i: (0, 0)),
        ],
        out_specs=[
            pl.BlockSpec((blk, OUT_CH), lambda i: (i, 0)),
            pl.BlockSpec((blk, 128), lambda i: (i, 0)),
        ],
        out_shape=[
            jax.ShapeDtypeStruct((N, OUT_CH), jnp.float32),
            jax.ShapeDtypeStruct((N, 128), jnp.float32),
        ],
    )(x, W, A2)


def _edge_body(h_hbm, asrc_hbm, adst_hbm, srci_hbm, dsti_hbm, zeros_hbm,
               msg_hbm, den_hbm, asrc_v, adst_v, sidx_v, didx_v, didxloc_v,
               rows_v, denloc_v, dloc_v, dexp_v, idv_v, acc_sh, accd_sh,
               gsem0, gsem1):
    c = lax.axis_index("c")
    s = lax.axis_index("s")
    nchunk = srci_hbm.shape[1] - 2
    r0 = s * RPT

    pltpu.sync_copy(asrc_hbm, asrc_v)
    pltpu.sync_copy(adst_hbm, adst_v)
    pltpu.sync_copy(srci_hbm.at[s], sidx_v)
    pltpu.sync_copy(dsti_hbm.at[s], didx_v)
    for t in range(DROWS // 16):
        idv_v[pl.ds(t * 16, 16)] = lax.iota(jnp.int32, 16) + t * 16
    zero16 = jnp.zeros((16,), jnp.float32)

    def pass_body(p, _):
        q = NC * p + c
        base = q * Q

        def zinit_body(z, _):
            pltpu.sync_copy(zeros_hbm, acc_sh.at[pl.ds(r0 + z * 32, 32)])
            return 0

        lax.fori_loop(0, RPT // 32, zinit_body, 0)
        pltpu.sync_copy(zeros_hbm.at[pl.ds(0, DRPT)],
                        accd_sh.at[pl.ds(s * DRPT, DRPT)])

        def zero_body(r, _):
            for t in range(8):
                denloc_v[r, pl.ds(t * 16, 16)] = zero16
            return 0

        lax.fori_loop(0, DROWS, zero_body, 0)
        plsc.subcore_barrier()

        def chunk_body(g, _):
            pltpu.async_copy(h_hbm.at[sidx_v.at[g]], rows_v.at[0], gsem0).wait()

            def group_body(j, _):
                si = sidx_v[g, pl.ds(j * LANES, LANES)]
                di = didx_v[g, pl.ds(j * LANES, LANES)]
                a = (plsc.load_gather(asrc_v, [si])
                     + plsc.load_gather(adst_v, [di]))
                a = jnp.where(a >= 0.0, a, 0.2 * a)
                e16 = jnp.exp(a)
                dil = di - base
                dil = jnp.where((dil >= 0) & (dil < Q), dil, Q)
                didxloc_v[0, pl.ds(j * LANES, LANES)] = dil
                plsc.addupdate_scatter(
                    denloc_v,
                    [lax.shift_right_logical(dil, 7),
                     lax.bitwise_and(dil, 127)],
                    e16)
                for l in range(LANES):
                    je = j * LANES + l
                    eb = jnp.full((16,), e16[l], jnp.float32)
                    for k in range(OUT_CH // 16):
                        rows_v[0, je, pl.ds(k * 16, 16)] = (
                            rows_v[0, je, pl.ds(k * 16, 16)] * eb)
                return 0

            lax.fori_loop(0, CH // LANES, group_body, 0)
            pltpu.sync_copy(rows_v.at[0], acc_sh.at[didxloc_v.at[0]], add=True)
            return 0

        lax.fori_loop(0, nchunk, chunk_body, 0)

        pltpu.sync_copy(denloc_v, accd_sh.at[idv_v], add=True)
        plsc.subcore_barrier()

        pltpu.sync_copy(acc_sh.at[pl.ds(r0, RPT)],
                        msg_hbm.at[q, pl.ds(r0, RPT)])
        pltpu.sync_copy(accd_sh, dloc_v)

        def exp_body(b, _):
            for t in range(2):
                o = r0 + b * 32 + t * 16
                dv = dloc_v[lax.shift_right_logical(o, 7),
                            pl.ds(lax.bitwise_and(o, 127), 16)]
                for l in range(LANES):
                    db = jnp.full((16,), dv[l], jnp.float32)
                    for k in range(OUT_CH // 16):
                        dexp_v[t * 16 + l, pl.ds(k * 16, 16)] = db
            pltpu.sync_copy(dexp_v, den_hbm.at[q, pl.ds(r0 + b * 32, 32)])
            return 0

        lax.fori_loop(0, RPT // 32, exp_body, 0)
        plsc.subcore_barrier()
        return 0

    lax.fori_loop(0, NP, pass_body, 0)


def _edge_pass(h, asrc, adst, srci, dsti):
    zeros = jnp.zeros((32, OUT_CH), jnp.float32)
    mesh = plsc.VectorSubcoreMesh(core_axis_name="c", subcore_axis_name="s")
    nchunk = srci.shape[1]
    kern = functools.partial(
        pl.kernel,
        mesh=mesh,
        compiler_params=pltpu.CompilerParams(needs_layout_passes=False),
        out_type=[
            jax.ShapeDtypeStruct((NQ, ACC_ROWS, OUT_CH), jnp.float32),
            jax.ShapeDtypeStruct((NQ, ACC_ROWS, OUT_CH), jnp.float32),
        ],
        scratch_types=[
            pltpu.VMEM((APAD,), jnp.float32),
            pltpu.VMEM((APAD,), jnp.float32),
            pltpu.VMEM((nchunk, CH), jnp.int32),
            pltpu.VMEM((nchunk, CH), jnp.int32),
            pltpu.VMEM((2, CH), jnp.int32),
            pltpu.VMEM((2, CH, OUT_CH), jnp.float32),
            pltpu.VMEM((DROWS, 128), jnp.float32),
            pltpu.VMEM((DROWS, 128), jnp.float32),
            pltpu.VMEM((32, OUT_CH), jnp.float32),
            pltpu.VMEM((DROWS,), jnp.int32),
            pltpu.VMEM_SHARED((ACC_ROWS, OUT_CH), jnp.float32),
            pltpu.VMEM_SHARED((DROWS, 128), jnp.float32),
            pltpu.SemaphoreType.DMA,
            pltpu.SemaphoreType.DMA,
        ],
    )(_edge_body)
    return kern(h, asrc, adst, srci, dsti, zeros)


def _fin_body(m_ref, d_ref, b_ref, o_ref):
    o_ref[...] = m_ref[...] / (d_ref[...] + 1e-16) + b_ref[...]


def _finalize(msg, den, bias):
    blk = 1024
    rows = msg.shape[0]
    grid = rows // blk
    return pl.pallas_call(
        _fin_body,
        grid=(grid,),
        in_specs=[
            pl.BlockSpec((blk, OUT_CH), lambda i: (i, 0)),
            pl.BlockSpec((blk, OUT_CH), lambda i: (i, 0)),
            pl.BlockSpec((1, OUT_CH), lambda i: (0, 0)),
        ],
        out_specs=pl.BlockSpec((blk, OUT_CH), lambda i: (i, 0)),
        out_shape=jax.ShapeDtypeStruct((rows, OUT_CH), jnp.float32),
    )(msg, den, bias)


def kernel(x, edge_index, W, att_src, att_dst, bias):
    A2 = jnp.zeros((OUT_CH, 128), jnp.float32)
    A2 = A2.at[:, 0].set(att_src[0].astype(jnp.float32))
    A2 = A2.at[:, 1].set(att_dst[0].astype(jnp.float32))

    h, a = _project(x, W, A2)
    asrc = jnp.pad(a[:, 0], (0, APAD - N))
    adst = jnp.pad(a[:, 1], (0, APAD - N))

    ei = edge_index.astype(jnp.int32)
    loop = jnp.arange(N, dtype=jnp.int32)
    e_tot = ei.shape[1] + N
    per = NS * CH
    nchunk = 2 * (-(-e_tot // (2 * per)))
    e_pad = nchunk * per
    src = jnp.concatenate([ei[0], loop, jnp.zeros((e_pad - e_tot,), jnp.int32)])
    dst = jnp.concatenate([ei[1], loop,
                           jnp.full((e_pad - e_tot,), N, jnp.int32)])
    srci = src.reshape(NS, nchunk, CH)
    dsti = dst.reshape(NS, nchunk, CH)
    srci = jnp.pad(srci, ((0, 0), (0, 2), (0, 0)))
    dsti = jnp.pad(dsti, ((0, 0), (0, 2), (0, 0)), constant_values=N)

    msg, den = _edge_pass(h, asrc, adst, srci, dsti)
    out = _finalize(msg.reshape(NQ * ACC_ROWS, OUT_CH),
                    den.reshape(NQ * ACC_ROWS, OUT_CH),
                    bias.reshape(1, OUT_CH))
    return jnp.concatenate(
        [out[i * ACC_ROWS:i * ACC_ROWS + Q] for i in range(NQ)])[:N]
